# baseline (speedup 1.0000x reference)
import functools

import jax
import jax.numpy as jnp
from jax import lax
from jax.experimental import pallas as pl
from jax.experimental.pallas import tpu as pltpu
from jax.experimental.pallas import tpu_sc as plsc

B = 16384
D = 32
NC = 2
NS = 16
L = 16
TPS = B // NS
EPS = 2 * TPS

FULL_BLK = 7812
SC0_BLK = 3907
WBLK = 240
BPT = WBLK // NS
NWIN = 17
WNODE = WBLK * 128
TAIL0 = FULL_BLK * 128
TAILN = 1000000 - TAIL0
TRASH = 2 * B

_mesh = plsc.VectorSubcoreMesh(
    core_axis_name="c", subcore_axis_name="s", num_cores=NC, num_subcores=NS
)
_params = pltpu.CompilerParams(needs_layout_passes=False)


@functools.partial(
    pl.kernel,
    out_type=jax.ShapeDtypeStruct(((2 * B + 1) * D,), jnp.float32),
    mesh=_mesh,
    compiler_params=_params,
    scratch_types=[
        pltpu.VMEM_SHARED((WBLK, 32, 128), jnp.float32),
        pltpu.VMEM((EPS,), jnp.int32),
        pltpu.VMEM((EPS,), jnp.int32),
        pltpu.VMEM((EPS + 2 * L,), jnp.int32),
        pltpu.VMEM((EPS + 2 * L,), jnp.int32),
        pltpu.SMEM((4,), jnp.int32),
        pltpu.VMEM((32, L), jnp.float32),
        pltpu.VMEM((L, 32), jnp.float32),
        pltpu.VMEM((32, TAILN), jnp.float32),
        pltpu.SemaphoreType.DMA,
        pltpu.SemaphoreType.DMA,
    ],
)
def _sc_extract(left_hbm, right_hbm, embt_hbm, vals_hbm, win_sh, nodes_v,
                pos_v, hnode_v, hpos_v, cnt_s, colT_v, rows_v, tail_v,
                sem, wsem):
    cid = lax.axis_index("c")
    sid = lax.axis_index("s")
    iota = lax.iota(jnp.int32, L)

    pltpu.sync_copy(left_hbm.at[pl.ds(sid * TPS, TPS)],
                    nodes_v.at[pl.ds(0, TPS)])
    pltpu.sync_copy(right_hbm.at[pl.ds(sid * TPS, TPS)],
                    nodes_v.at[pl.ds(TPS, TPS)])
    for k in range(TPS // L):
        pos_v[pl.ds(k * L, L)] = sid * TPS + k * L + iota
    for k in range(TPS // L):
        pos_v[pl.ds(TPS + k * L, L)] = B + sid * TPS + k * L + iota

    pltpu.sync_copy(embt_hbm.at[:, pl.ds(TAIL0, TAILN)], tail_v)

    half_begin = cid * SC0_BLK
    half_end = SC0_BLK + cid * (FULL_BLK - SC0_BLK)

    cnt_s[0] = 0

    def scan_entries(w0n, w1n, sent_node):
        def scan(k, _):
            nodes = nodes_v[pl.ds(k * L, L)]
            m = (nodes >= w0n) & (nodes < w1n)
            p = pos_v[pl.ds(k * L, L)]
            c = cnt_s[0]
            mi = m.astype(jnp.int32)
            slots = c + plsc.cumsum(mi) - mi
            plsc.store_scatter(hnode_v, [slots], nodes, mask=m)
            plsc.store_scatter(hpos_v, [slots], p, mask=m)
            cnt_s[0] = c + plsc.all_reduce_population_count(m)[0]
            return _
        lax.fori_loop(0, EPS // L, scan, None)

        c = cnt_s[0]
        a0 = pl.multiple_of(lax.bitwise_and(c, -L), L)
        keep = (a0 + iota) < c
        nv = hnode_v[pl.ds(a0, L)]
        pv = hpos_v[pl.ds(a0, L)]
        hnode_v[pl.ds(a0, L)] = jnp.where(keep, nv, sent_node)
        hpos_v[pl.ds(a0, L)] = jnp.where(keep, pv, TRASH)
        cnt_s[0] = lax.bitwise_and(c + (L - 1), -L)

    def process_hits(lo, hi, w0n, from_tail):
        def chunk16(q, _):
            cb = pl.multiple_of(q * L, L)
            hn16 = hnode_v[pl.ds(cb, L)]
            hp16 = hpos_v[pl.ds(cb, L)]

            if not from_tail:
                rel16 = hn16 - w0n
                for j in range(L):
                    rel = rel16[j]
                    blk = lax.shift_right_logical(rel, 7)
                    lane = lax.bitwise_and(rel, 127)
                    pltpu.async_copy(
                        win_sh.at[blk, :, pl.ds(lane, 1)],
                        colT_v.at[:, pl.ds(j, 1)], sem)
                for j in range(L):
                    pltpu.make_async_copy(
                        win_sh.at[0, :, pl.ds(0, 1)],
                        colT_v.at[:, pl.ds(j, 1)], sem).wait()
                for j in range(L):
                    cols = jnp.full((L,), j, jnp.int32)
                    ja = plsc.load_gather(colT_v, [iota, cols])
                    jb = plsc.load_gather(colT_v, [L + iota, cols])
                    rows_v[j, pl.ds(0, L)] = ja
                    rows_v[j, pl.ds(L, L)] = jb
            else:
                for j in range(L):
                    lanes = jnp.full((L,), hn16[j] - TAIL0, jnp.int32)
                    ja = plsc.load_gather(tail_v, [iota, lanes])
                    jb = plsc.load_gather(tail_v, [L + iota, lanes])
                    rows_v[j, pl.ds(0, L)] = ja
                    rows_v[j, pl.ds(L, L)] = jb

            for j in range(L):
                pltpu.async_copy(
                    rows_v.at[j], vals_hbm.at[pl.ds(hp16[j] * D, D)], wsem)
            for j in range(L):
                pltpu.make_async_copy(
                    rows_v.at[j], vals_hbm.at[pl.ds(0, D)], wsem).wait()
            return _

        lax.fori_loop(lax.div(lo, L), lax.div(hi, L), chunk16, None)

    def win_body(w, _):
        base_blk = half_begin + w * WBLK + sid * BPT
        for j in range(BPT):
            blk = base_blk + j

            @pl.when(blk < half_end)
            def _fire(blk=blk, j=j):
                pltpu.async_copy(
                    embt_hbm.at[:, pl.ds(blk * 128, 128)],
                    win_sh.at[sid * BPT + j], sem)
        for j in range(BPT):
            blk = base_blk + j

            @pl.when(blk < half_end)
            def _drain(j=j):
                pltpu.make_async_copy(
                    embt_hbm.at[:, pl.ds(0, 128)],
                    win_sh.at[sid * BPT + j], sem).wait()
        plsc.subcore_barrier()

        w0n = (half_begin + w * WBLK) * 128
        w1n = jnp.minimum(w0n + WNODE, half_end * 128)
        lo = cnt_s[0]
        scan_entries(w0n, w1n, w0n)
        hi = cnt_s[0]
        process_hits(lo, hi, w0n, False)
        plsc.subcore_barrier()
        return _

    lax.fori_loop(0, NWIN, win_body, None)

    @pl.when(cid == 1)
    def _tail():
        lo = cnt_s[0]
        scan_entries(TAIL0, 1000000, TAIL0)
        hi = cnt_s[0]
        process_hits(lo, hi, 0, True)


@functools.partial(
    pl.kernel,
    out_type=jax.ShapeDtypeStruct((B,), jnp.float32),
    mesh=_mesh,
    compiler_params=_params,
    scratch_types=[
        pltpu.VMEM((512 * D,), jnp.float32),
        pltpu.VMEM((512 * D,), jnp.float32),
        pltpu.VMEM((512,), jnp.float32),
    ],
)
def _sc_dot(vals_hbm, out_hbm, lv_v, rv_v, out_v):
    cid = lax.axis_index("c")
    sid = lax.axis_index("s")
    wid = sid * NC + cid
    b0 = wid * 512
    iota = lax.iota(jnp.int32, L)

    pltpu.sync_copy(vals_hbm.at[pl.ds(b0 * D, 512 * D)], lv_v)
    pltpu.sync_copy(vals_hbm.at[pl.ds((B + b0) * D, 512 * D)], rv_v)

    def g_body(g, _):
        base = (g * L + iota) * D
        acc = jnp.zeros((L,), jnp.float32)
        for d in range(D):
            lval = plsc.load_gather(lv_v, [base + d])
            rval = plsc.load_gather(rv_v, [base + d])
            acc = acc + lval * rval
        out_v[pl.ds(g * L, L)] = acc
        return _

    lax.fori_loop(0, 512 // L, g_body, None)
    pltpu.sync_copy(out_v, out_hbm.at[pl.ds(b0, 512)])


def kernel(triplets, node_emb, vars):
    left = triplets[:, 0]
    right = triplets[:, 2]
    vals = _sc_extract(left, right, node_emb.T)
    return _sc_dot(vals)


# speedup vs baseline: 1.0776x; 1.0776x over previous
import functools

import jax
import jax.numpy as jnp
from jax import lax
from jax.experimental import pallas as pl
from jax.experimental.pallas import tpu as pltpu
from jax.experimental.pallas import tpu_sc as plsc

B = 16384
D = 32
NC = 2
NS = 16
L = 16
TPS = B // NS
EPS = 2 * TPS

FULL_BLK = 7812
SC0_BLK = 3907
WBLK = 240
BPT = WBLK // NS
NWIN = 17
WNODE = WBLK * 128
TAIL0 = FULL_BLK * 128
TAILN = 1000000 - TAIL0
TRASH = 2 * B

_mesh = plsc.VectorSubcoreMesh(
    core_axis_name="c", subcore_axis_name="s", num_cores=NC, num_subcores=NS
)
_params = pltpu.CompilerParams(needs_layout_passes=False)


@functools.partial(
    pl.kernel,
    out_type=jax.ShapeDtypeStruct(((2 * B + 1) * D,), jnp.float32),
    mesh=_mesh,
    compiler_params=_params,
    scratch_types=[
        pltpu.VMEM_SHARED((WBLK, 32, 128), jnp.float32),
        pltpu.VMEM((EPS,), jnp.int32),
        pltpu.VMEM((EPS,), jnp.int32),
        pltpu.VMEM((EPS + 2 * L,), jnp.int32),
        pltpu.VMEM((EPS + 2 * L,), jnp.int32),
        pltpu.SMEM((4,), jnp.int32),
        pltpu.VMEM((32, L), jnp.float32),
        pltpu.VMEM((L, 32), jnp.float32),
        pltpu.VMEM((32, TAILN), jnp.float32),
        pltpu.SemaphoreType.DMA,
        pltpu.SemaphoreType.DMA,
    ],
)
def _sc_extract(left_hbm, right_hbm, embt_hbm, vals_hbm, win_sh, nodes_v,
                pos_v, hnode_v, hpos_v, cnt_s, colT_v, rows_v, tail_v,
                sem, wsem):
    cid = lax.axis_index("c")
    sid = lax.axis_index("s")
    iota = lax.iota(jnp.int32, L)

    pltpu.sync_copy(left_hbm.at[pl.ds(sid * TPS, TPS)],
                    nodes_v.at[pl.ds(0, TPS)])
    pltpu.sync_copy(right_hbm.at[pl.ds(sid * TPS, TPS)],
                    nodes_v.at[pl.ds(TPS, TPS)])
    for k in range(TPS // L):
        pos_v[pl.ds(k * L, L)] = sid * TPS + k * L + iota
    for k in range(TPS // L):
        pos_v[pl.ds(TPS + k * L, L)] = B + sid * TPS + k * L + iota

    pltpu.sync_copy(embt_hbm.at[:, pl.ds(TAIL0, TAILN)], tail_v)

    half_begin = cid * SC0_BLK
    half_end = SC0_BLK + cid * (FULL_BLK - SC0_BLK)

    cnt_s[0] = 0

    def scan_entries(w0n, w1n, sent_node):
        def scan(k, c):
            nodes = nodes_v[pl.ds(k * L, L)]
            m = (nodes >= w0n) & (nodes < w1n)
            p = pos_v[pl.ds(k * L, L)]
            mi = m.astype(jnp.int32)
            cs = plsc.cumsum(mi)
            slots = c + cs - mi
            plsc.store_scatter(hnode_v, [slots], nodes, mask=m)
            plsc.store_scatter(hpos_v, [slots], p, mask=m)
            return c + cs[15]
        c = lax.fori_loop(0, EPS // L, scan, cnt_s[0])

        a0 = pl.multiple_of(lax.bitwise_and(c, -L), L)
        keep = (a0 + iota) < c
        nv = hnode_v[pl.ds(a0, L)]
        pv = hpos_v[pl.ds(a0, L)]
        hnode_v[pl.ds(a0, L)] = jnp.where(keep, nv, sent_node)
        hpos_v[pl.ds(a0, L)] = jnp.where(keep, pv, TRASH)
        cnt_s[0] = lax.bitwise_and(c + (L - 1), -L)

    def process_hits(lo, hi, w0n, from_tail):
        def chunk16(q, _):
            cb = pl.multiple_of(q * L, L)
            hn16 = hnode_v[pl.ds(cb, L)]
            hp16 = hpos_v[pl.ds(cb, L)]

            if not from_tail:
                rel16 = hn16 - w0n
                for j in range(L):
                    rel = rel16[j]
                    blk = lax.shift_right_logical(rel, 7)
                    lane = lax.bitwise_and(rel, 127)
                    pltpu.async_copy(
                        win_sh.at[blk, :, pl.ds(lane, 1)],
                        colT_v.at[:, pl.ds(j, 1)], sem)
                for j in range(L):
                    pltpu.make_async_copy(
                        win_sh.at[0, :, pl.ds(0, 1)],
                        colT_v.at[:, pl.ds(j, 1)], sem).wait()
                for j in range(L):
                    cols = jnp.full((L,), j, jnp.int32)
                    ja = plsc.load_gather(colT_v, [iota, cols])
                    jb = plsc.load_gather(colT_v, [L + iota, cols])
                    rows_v[j, pl.ds(0, L)] = ja
                    rows_v[j, pl.ds(L, L)] = jb
            else:
                for j in range(L):
                    lanes = jnp.full((L,), hn16[j] - TAIL0, jnp.int32)
                    ja = plsc.load_gather(tail_v, [iota, lanes])
                    jb = plsc.load_gather(tail_v, [L + iota, lanes])
                    rows_v[j, pl.ds(0, L)] = ja
                    rows_v[j, pl.ds(L, L)] = jb

            for j in range(L):
                pltpu.async_copy(
                    rows_v.at[j], vals_hbm.at[pl.ds(hp16[j] * D, D)], wsem)
            for j in range(L):
                pltpu.make_async_copy(
                    rows_v.at[j], vals_hbm.at[pl.ds(0, D)], wsem).wait()
            return _

        lax.fori_loop(lax.div(lo, L), lax.div(hi, L), chunk16, None)

    def win_body(w, _):
        base_blk = half_begin + w * WBLK + sid * BPT
        for j in range(BPT):
            blk = base_blk + j

            @pl.when(blk < half_end)
            def _fire(blk=blk, j=j):
                pltpu.async_copy(
                    embt_hbm.at[:, pl.ds(blk * 128, 128)],
                    win_sh.at[sid * BPT + j], sem)

        w0n = (half_begin + w * WBLK) * 128
        w1n = jnp.minimum(w0n + WNODE, half_end * 128)
        lo = cnt_s[0]
        scan_entries(w0n, w1n, w0n)
        hi = cnt_s[0]

        for j in range(BPT):
            blk = base_blk + j

            @pl.when(blk < half_end)
            def _drain(j=j):
                pltpu.make_async_copy(
                    embt_hbm.at[:, pl.ds(0, 128)],
                    win_sh.at[sid * BPT + j], sem).wait()
        plsc.subcore_barrier()

        process_hits(lo, hi, w0n, False)
        plsc.subcore_barrier()
        return _

    lax.fori_loop(0, NWIN, win_body, None)

    @pl.when(cid == 1)
    def _tail():
        lo = cnt_s[0]
        scan_entries(TAIL0, 1000000, TAIL0)
        hi = cnt_s[0]
        process_hits(lo, hi, 0, True)


@functools.partial(
    pl.kernel,
    out_type=jax.ShapeDtypeStruct((B,), jnp.float32),
    mesh=_mesh,
    compiler_params=_params,
    scratch_types=[
        pltpu.VMEM((512 * D,), jnp.float32),
        pltpu.VMEM((512 * D,), jnp.float32),
        pltpu.VMEM((512,), jnp.float32),
    ],
)
def _sc_dot(vals_hbm, out_hbm, lv_v, rv_v, out_v):
    cid = lax.axis_index("c")
    sid = lax.axis_index("s")
    wid = sid * NC + cid
    b0 = wid * 512
    iota = lax.iota(jnp.int32, L)

    pltpu.sync_copy(vals_hbm.at[pl.ds(b0 * D, 512 * D)], lv_v)
    pltpu.sync_copy(vals_hbm.at[pl.ds((B + b0) * D, 512 * D)], rv_v)

    def g_body(g, _):
        base = (g * L + iota) * D
        acc = jnp.zeros((L,), jnp.float32)
        for d in range(D):
            lval = plsc.load_gather(lv_v, [base + d])
            rval = plsc.load_gather(rv_v, [base + d])
            acc = acc + lval * rval
        out_v[pl.ds(g * L, L)] = acc
        return _

    lax.fori_loop(0, 512 // L, g_body, None)
    pltpu.sync_copy(out_v, out_hbm.at[pl.ds(b0, 512)])


def kernel(triplets, node_emb, vars):
    left = triplets[:, 0]
    right = triplets[:, 2]
    vals = _sc_extract(left, right, node_emb.T)
    return _sc_dot(vals)


# speedup vs baseline: 1.0821x; 1.0041x over previous
import functools

import jax
import jax.numpy as jnp
from jax import lax
from jax.experimental import pallas as pl
from jax.experimental.pallas import tpu as pltpu
from jax.experimental.pallas import tpu_sc as plsc

B = 16384
D = 32
NC = 2
NS = 16
L = 16
TPS = B // NS
EPS = 2 * TPS

FULL_BLK = 7812
SC0_BLK = 3907
WBLK = 240
BPT = WBLK // NS
NWIN = 17
WNODE = WBLK * 128
TAIL0 = FULL_BLK * 128
TAILN = 1000000 - TAIL0
TRASH = 2 * B

_mesh = plsc.VectorSubcoreMesh(
    core_axis_name="c", subcore_axis_name="s", num_cores=NC, num_subcores=NS
)
_params = pltpu.CompilerParams(needs_layout_passes=False)


@functools.partial(
    pl.kernel,
    out_type=jax.ShapeDtypeStruct(((2 * B + 1) * D,), jnp.float32),
    mesh=_mesh,
    compiler_params=_params,
    scratch_types=[
        pltpu.VMEM_SHARED((WBLK, 32, 128), jnp.float32),
        pltpu.VMEM((EPS,), jnp.int32),
        pltpu.VMEM((EPS,), jnp.int32),
        pltpu.VMEM((EPS + 2 * L,), jnp.int32),
        pltpu.VMEM((EPS + 2 * L,), jnp.int32),
        pltpu.SMEM((4,), jnp.int32),
        pltpu.VMEM((2, 32, L), jnp.float32),
        pltpu.VMEM((L, 32), jnp.float32),
        pltpu.VMEM((32, TAILN), jnp.float32),
        pltpu.SemaphoreType.DMA,
        pltpu.SemaphoreType.DMA,
    ],
)
def _sc_extract(left_hbm, right_hbm, embt_hbm, vals_hbm, win_sh, nodes_v,
                pos_v, hnode_v, hpos_v, cnt_s, colT_v, rows_v, tail_v,
                sem, wsem):
    cid = lax.axis_index("c")
    sid = lax.axis_index("s")
    iota = lax.iota(jnp.int32, L)

    pltpu.sync_copy(left_hbm.at[pl.ds(sid * TPS, TPS)],
                    nodes_v.at[pl.ds(0, TPS)])
    pltpu.sync_copy(right_hbm.at[pl.ds(sid * TPS, TPS)],
                    nodes_v.at[pl.ds(TPS, TPS)])
    for k in range(TPS // L):
        pos_v[pl.ds(k * L, L)] = sid * TPS + k * L + iota
    for k in range(TPS // L):
        pos_v[pl.ds(TPS + k * L, L)] = B + sid * TPS + k * L + iota

    pltpu.sync_copy(embt_hbm.at[:, pl.ds(TAIL0, TAILN)], tail_v)

    half_begin = cid * SC0_BLK
    half_end = SC0_BLK + cid * (FULL_BLK - SC0_BLK)

    cnt_s[0] = 0

    def scan_entries(w0n, w1n, sent_node):
        def scan(k, c):
            nodes = nodes_v[pl.ds(k * L, L)]
            m = (nodes >= w0n) & (nodes < w1n)
            p = pos_v[pl.ds(k * L, L)]
            mi = m.astype(jnp.int32)
            cs = plsc.cumsum(mi)
            slots = c + cs - mi
            plsc.store_scatter(hnode_v, [slots], nodes, mask=m)
            plsc.store_scatter(hpos_v, [slots], p, mask=m)
            return c + cs[15]
        c = lax.fori_loop(0, EPS // L, scan, cnt_s[0])

        a0 = pl.multiple_of(lax.bitwise_and(c, -L), L)
        keep = (a0 + iota) < c
        nv = hnode_v[pl.ds(a0, L)]
        pv = hpos_v[pl.ds(a0, L)]
        hnode_v[pl.ds(a0, L)] = jnp.where(keep, nv, sent_node)
        hpos_v[pl.ds(a0, L)] = jnp.where(keep, pv, TRASH)
        cnt_s[0] = lax.bitwise_and(c + (L - 1), -L)

    def finish_chunk(q, buf, w0n, from_tail):
        cb = pl.multiple_of(q * L, L)
        hn16 = hnode_v[pl.ds(cb, L)]
        hp16 = hpos_v[pl.ds(cb, L)]

        if not from_tail:
            for j in range(L):
                pltpu.make_async_copy(
                    win_sh.at[0, :, pl.ds(0, 1)],
                    colT_v.at[buf, :, pl.ds(j, 1)], sem).wait()
            for j in range(L):
                cols = jnp.full((L,), j, jnp.int32)
                bufs = jnp.full((L,), buf, jnp.int32)
                ja = plsc.load_gather(colT_v, [bufs, iota, cols])
                jb = plsc.load_gather(colT_v, [bufs, L + iota, cols])
                rows_v[j, pl.ds(0, L)] = ja
                rows_v[j, pl.ds(L, L)] = jb
        else:
            for j in range(L):
                lanes = jnp.full((L,), hn16[j] - TAIL0, jnp.int32)
                ja = plsc.load_gather(tail_v, [iota, lanes])
                jb = plsc.load_gather(tail_v, [L + iota, lanes])
                rows_v[j, pl.ds(0, L)] = ja
                rows_v[j, pl.ds(L, L)] = jb

        for j in range(L):
            pltpu.async_copy(
                rows_v.at[j], vals_hbm.at[pl.ds(hp16[j] * D, D)], wsem)
        for j in range(L):
            pltpu.make_async_copy(
                rows_v.at[j], vals_hbm.at[pl.ds(0, D)], wsem).wait()

    def process_hits(lo, hi, w0n, from_tail):
        lo_div = lax.div(lo, L)
        hi_div = lax.div(hi, L)

        if from_tail:
            def chunk_t(q, _):
                finish_chunk(q, 0, w0n, True)
                return _
            lax.fori_loop(lo_div, hi_div, chunk_t, None)
            return

        def chunk16(q, _):
            buf = lax.bitwise_and(q, 1)
            cb = pl.multiple_of(q * L, L)
            hn16 = hnode_v[pl.ds(cb, L)]
            rel16 = hn16 - w0n
            for j in range(L):
                rel = rel16[j]
                blk = lax.shift_right_logical(rel, 7)
                lane = lax.bitwise_and(rel, 127)
                pltpu.async_copy(
                    win_sh.at[blk, :, pl.ds(lane, 1)],
                    colT_v.at[buf, :, pl.ds(j, 1)], sem)

            @pl.when(q > lo_div)
            def _prev():
                finish_chunk(q - 1, 1 - buf, w0n, False)
            return _

        lax.fori_loop(lo_div, hi_div, chunk16, None)

        @pl.when(hi_div > lo_div)
        def _last():
            finish_chunk(hi_div - 1,
                         lax.bitwise_and(hi_div - 1, 1), w0n, False)

    def win_body(w, _):
        base_blk = half_begin + w * WBLK + sid * BPT
        for j in range(BPT):
            blk = base_blk + j

            @pl.when(blk < half_end)
            def _fire(blk=blk, j=j):
                pltpu.async_copy(
                    embt_hbm.at[:, pl.ds(blk * 128, 128)],
                    win_sh.at[sid * BPT + j], sem)

        w0n = (half_begin + w * WBLK) * 128
        w1n = jnp.minimum(w0n + WNODE, half_end * 128)
        lo = cnt_s[0]
        scan_entries(w0n, w1n, w0n)
        hi = cnt_s[0]

        for j in range(BPT):
            blk = base_blk + j

            @pl.when(blk < half_end)
            def _drain(j=j):
                pltpu.make_async_copy(
                    embt_hbm.at[:, pl.ds(0, 128)],
                    win_sh.at[sid * BPT + j], sem).wait()
        plsc.subcore_barrier()

        process_hits(lo, hi, w0n, False)
        plsc.subcore_barrier()
        return _

    lax.fori_loop(0, NWIN, win_body, None)

    @pl.when(cid == 1)
    def _tail():
        lo = cnt_s[0]
        scan_entries(TAIL0, 1000000, TAIL0)
        hi = cnt_s[0]
        process_hits(lo, hi, 0, True)


@functools.partial(
    pl.kernel,
    out_type=jax.ShapeDtypeStruct((B,), jnp.float32),
    mesh=_mesh,
    compiler_params=_params,
    scratch_types=[
        pltpu.VMEM((512 * D,), jnp.float32),
        pltpu.VMEM((512 * D,), jnp.float32),
        pltpu.VMEM((512,), jnp.float32),
    ],
)
def _sc_dot(vals_hbm, out_hbm, lv_v, rv_v, out_v):
    cid = lax.axis_index("c")
    sid = lax.axis_index("s")
    wid = sid * NC + cid
    b0 = wid * 512
    iota = lax.iota(jnp.int32, L)

    pltpu.sync_copy(vals_hbm.at[pl.ds(b0 * D, 512 * D)], lv_v)
    pltpu.sync_copy(vals_hbm.at[pl.ds((B + b0) * D, 512 * D)], rv_v)

    def g_body(g, _):
        base = (g * L + iota) * D
        acc = jnp.zeros((L,), jnp.float32)
        for d in range(D):
            lval = plsc.load_gather(lv_v, [base + d])
            rval = plsc.load_gather(rv_v, [base + d])
            acc = acc + lval * rval
        out_v[pl.ds(g * L, L)] = acc
        return _

    lax.fori_loop(0, 512 // L, g_body, None)
    pltpu.sync_copy(out_v, out_hbm.at[pl.ds(b0, 512)])


def kernel(triplets, node_emb, vars):
    left = triplets[:, 0]
    right = triplets[:, 2]
    vals = _sc_extract(left, right, node_emb.T)
    return _sc_dot(vals)


# speedup vs baseline: 1.1660x; 1.0775x over previous
import functools

import jax
import jax.numpy as jnp
from jax import lax
from jax.experimental import pallas as pl
from jax.experimental.pallas import tpu as pltpu
from jax.experimental.pallas import tpu_sc as plsc

B = 16384
D = 32
NC = 2
NS = 16
L = 16
TPS = B // NS
EPS = 2 * TPS

FULL_BLK = 7812
SC0_BLK = 3907
WBLK = 208
BPT = WBLK // NS
NWIN = 19
WNODE = WBLK * 128
TAIL0 = FULL_BLK * 128
TAILN = 1000000 - TAIL0
TRASH = 2 * B

_mesh = plsc.VectorSubcoreMesh(
    core_axis_name="c", subcore_axis_name="s", num_cores=NC, num_subcores=NS
)
_params = pltpu.CompilerParams(needs_layout_passes=False)


@functools.partial(
    pl.kernel,
    out_type=jax.ShapeDtypeStruct(((2 * B + 1) * D,), jnp.float32),
    mesh=_mesh,
    compiler_params=_params,
    scratch_types=[
        pltpu.VMEM_SHARED((2, WBLK, 32, 128), jnp.float32),
        pltpu.VMEM((EPS,), jnp.int32),
        pltpu.VMEM((EPS,), jnp.int32),
        pltpu.VMEM((EPS + 2 * L,), jnp.int32),
        pltpu.VMEM((EPS + 2 * L,), jnp.int32),
        pltpu.SMEM((4,), jnp.int32),
        pltpu.VMEM((32, L), jnp.float32),
        pltpu.VMEM((L, 32), jnp.float32),
        pltpu.VMEM((32, TAILN), jnp.float32),
        pltpu.SemaphoreType.DMA,
        pltpu.SemaphoreType.DMA,
        pltpu.SemaphoreType.DMA,
        pltpu.SemaphoreType.DMA,
    ],
)
def _sc_extract(left_hbm, right_hbm, embt_hbm, vals_hbm, win_sh, nodes_v,
                pos_v, hnode_v, hpos_v, cnt_s, colT_v, rows_v, tail_v,
                sem, wsem, wsem0, wsem1):
    cid = lax.axis_index("c")
    sid = lax.axis_index("s")
    iota = lax.iota(jnp.int32, L)

    pltpu.sync_copy(left_hbm.at[pl.ds(sid * TPS, TPS)],
                    nodes_v.at[pl.ds(0, TPS)])
    pltpu.sync_copy(right_hbm.at[pl.ds(sid * TPS, TPS)],
                    nodes_v.at[pl.ds(TPS, TPS)])
    for k in range(TPS // L):
        pos_v[pl.ds(k * L, L)] = sid * TPS + k * L + iota
    for k in range(TPS // L):
        pos_v[pl.ds(TPS + k * L, L)] = B + sid * TPS + k * L + iota

    pltpu.sync_copy(embt_hbm.at[:, pl.ds(TAIL0, TAILN)], tail_v)

    half_begin = cid * SC0_BLK
    half_end = SC0_BLK + cid * (FULL_BLK - SC0_BLK)

    cnt_s[0] = 0

    def scan_entries(w0n, w1n, sent_node):
        def scan(k, c):
            nodes = nodes_v[pl.ds(k * L, L)]
            m = (nodes >= w0n) & (nodes < w1n)
            p = pos_v[pl.ds(k * L, L)]
            mi = m.astype(jnp.int32)
            cs = plsc.cumsum(mi)
            slots = c + cs - mi
            plsc.store_scatter(hnode_v, [slots], nodes, mask=m)
            plsc.store_scatter(hpos_v, [slots], p, mask=m)
            return c + cs[15]
        c = lax.fori_loop(0, EPS // L, scan, cnt_s[0])

        a0 = pl.multiple_of(lax.bitwise_and(c, -L), L)
        keep = (a0 + iota) < c
        nv = hnode_v[pl.ds(a0, L)]
        pv = hpos_v[pl.ds(a0, L)]
        hnode_v[pl.ds(a0, L)] = jnp.where(keep, nv, sent_node)
        hpos_v[pl.ds(a0, L)] = jnp.where(keep, pv, TRASH)
        cnt_s[0] = lax.bitwise_and(c + (L - 1), -L)

    def process_hits(lo, hi, w0n, wbuf, from_tail):
        def chunk16(q, _):
            cb = pl.multiple_of(q * L, L)
            hn16 = hnode_v[pl.ds(cb, L)]
            hp16 = hpos_v[pl.ds(cb, L)]

            if not from_tail:
                rel16 = hn16 - w0n
                for j in range(L):
                    rel = rel16[j]
                    blk = lax.shift_right_logical(rel, 7)
                    lane = lax.bitwise_and(rel, 127)
                    pltpu.async_copy(
                        win_sh.at[wbuf, blk, :, pl.ds(lane, 1)],
                        colT_v.at[:, pl.ds(j, 1)], sem)
                for j in range(L):
                    pltpu.make_async_copy(
                        win_sh.at[0, 0, :, pl.ds(0, 1)],
                        colT_v.at[:, pl.ds(j, 1)], sem).wait()
                for j in range(L):
                    cols = jnp.full((L,), j, jnp.int32)
                    ja = plsc.load_gather(colT_v, [iota, cols])
                    jb = plsc.load_gather(colT_v, [L + iota, cols])
                    rows_v[j, pl.ds(0, L)] = ja
                    rows_v[j, pl.ds(L, L)] = jb
            else:
                for j in range(L):
                    lanes = jnp.full((L,), hn16[j] - TAIL0, jnp.int32)
                    ja = plsc.load_gather(tail_v, [iota, lanes])
                    jb = plsc.load_gather(tail_v, [L + iota, lanes])
                    rows_v[j, pl.ds(0, L)] = ja
                    rows_v[j, pl.ds(L, L)] = jb

            for j in range(L):
                pltpu.async_copy(
                    rows_v.at[j], vals_hbm.at[pl.ds(hp16[j] * D, D)], wsem)
            for j in range(L):
                pltpu.make_async_copy(
                    rows_v.at[j], vals_hbm.at[pl.ds(0, D)], wsem).wait()
            return _

        lax.fori_loop(lax.div(lo, L), lax.div(hi, L), chunk16, None)

    def fire_window(w, wbuf, wsemx):
        base_blk = half_begin + w * WBLK + sid * BPT
        for j in range(BPT):
            blk = base_blk + j

            @pl.when(blk < half_end)
            def _fire(blk=blk, j=j):
                pltpu.async_copy(
                    embt_hbm.at[:, pl.ds(blk * 128, 128)],
                    win_sh.at[wbuf, sid * BPT + j], wsemx)

    def drain_window(w, wbuf, wsemx):
        base_blk = half_begin + w * WBLK + sid * BPT
        for j in range(BPT):
            blk = base_blk + j

            @pl.when(blk < half_end)
            def _drain(j=j):
                pltpu.make_async_copy(
                    embt_hbm.at[:, pl.ds(0, 128)],
                    win_sh.at[wbuf, sid * BPT + j], wsemx).wait()

    fire_window(0, 0, wsem0)

    def win_body(w, _):
        par = lax.bitwise_and(w, 1)
        nxt = 1 - par

        @pl.when((w + 1 < NWIN) & (nxt == 1))
        def _f1():
            fire_window(w + 1, 1, wsem1)

        @pl.when((w + 1 < NWIN) & (nxt == 0))
        def _f0():
            fire_window(w + 1, 0, wsem0)

        w0n = (half_begin + w * WBLK) * 128
        w1n = jnp.minimum(w0n + WNODE, half_end * 128)
        lo = cnt_s[0]
        scan_entries(w0n, w1n, w0n)
        hi = cnt_s[0]

        @pl.when(par == 0)
        def _d0():
            drain_window(w, 0, wsem0)

        @pl.when(par == 1)
        def _d1():
            drain_window(w, 1, wsem1)
        plsc.subcore_barrier()

        process_hits(lo, hi, w0n, par, False)
        plsc.subcore_barrier()
        return _

    lax.fori_loop(0, NWIN, win_body, None)

    @pl.when(cid == 1)
    def _tail():
        lo = cnt_s[0]
        scan_entries(TAIL0, 1000000, TAIL0)
        hi = cnt_s[0]
        process_hits(lo, hi, 0, 0, True)


@functools.partial(
    pl.kernel,
    out_type=jax.ShapeDtypeStruct((B,), jnp.float32),
    mesh=_mesh,
    compiler_params=_params,
    scratch_types=[
        pltpu.VMEM((512 * D,), jnp.float32),
        pltpu.VMEM((512 * D,), jnp.float32),
        pltpu.VMEM((512,), jnp.float32),
    ],
)
def _sc_dot(vals_hbm, out_hbm, lv_v, rv_v, out_v):
    cid = lax.axis_index("c")
    sid = lax.axis_index("s")
    wid = sid * NC + cid
    b0 = wid * 512
    iota = lax.iota(jnp.int32, L)

    pltpu.sync_copy(vals_hbm.at[pl.ds(b0 * D, 512 * D)], lv_v)
    pltpu.sync_copy(vals_hbm.at[pl.ds((B + b0) * D, 512 * D)], rv_v)

    def g_body(g, _):
        base = (g * L + iota) * D
        acc = jnp.zeros((L,), jnp.float32)
        for d in range(D):
            lval = plsc.load_gather(lv_v, [base + d])
            rval = plsc.load_gather(rv_v, [base + d])
            acc = acc + lval * rval
        out_v[pl.ds(g * L, L)] = acc
        return _

    lax.fori_loop(0, 512 // L, g_body, None)
    pltpu.sync_copy(out_v, out_hbm.at[pl.ds(b0, 512)])


def kernel(triplets, node_emb, vars):
    left = triplets[:, 0]
    right = triplets[:, 2]
    vals = _sc_extract(left, right, node_emb.T)
    return _sc_dot(vals)
